# baseline (speedup 1.0000x reference)
import jax
import jax.numpy as jnp
from jax.experimental import pallas as pl
from jax.experimental.pallas import tpu as pltpu

_BH = 200
_BR = 2 * _BH


def _gcn_body(x_ref, adja_ref, adjb_ref, w1_ref, w2_ref, wout_ref,
              x3_ref, y_ref, s1_ref, s2_ref, s2b_ref):
    p = pl.program_id(0)
    i = pl.program_id(1)

    @pl.when(jnp.logical_and(p == 0, i == 0))
    def _():
        s1 = jnp.dot(x_ref[...], w1_ref[...],
                     preferred_element_type=jnp.float32)
        s1_ref[...] = s1.astype(jnp.bfloat16)

    a = adja_ref[...].astype(jnp.bfloat16)
    b = adjb_ref[...].astype(jnp.bfloat16)

    @pl.when(p == 0)
    def _():
        w2 = w2_ref[...].astype(jnp.bfloat16)
        ha = jnp.dot(a, s1_ref[...], preferred_element_type=jnp.float32)
        hb = jnp.dot(b, s1_ref[...], preferred_element_type=jnp.float32)
        x1a = jnp.where(ha >= 0, ha, 0.01 * ha)
        x1b = jnp.where(hb >= 0, hb, 0.01 * hb)
        s2a = jnp.dot(x1a.astype(jnp.bfloat16), w2,
                      preferred_element_type=jnp.float32)
        s2b = jnp.dot(x1b.astype(jnp.bfloat16), w2,
                      preferred_element_type=jnp.float32)
        s2_ref[pl.ds(i * _BR, _BH), :] = s2a
        s2_ref[pl.ds(i * _BR + _BH, _BH), :] = s2b
        x3_ref[...] = jnp.zeros_like(x3_ref)
        y_ref[...] = jnp.zeros_like(y_ref)

    @pl.when(jnp.logical_and(p == 1, i == 0))
    def _():
        s2b_ref[...] = s2_ref[...].astype(jnp.bfloat16)

    @pl.when(p == 1)
    def _():
        wout = wout_ref[...].astype(jnp.bfloat16)
        x3a = jnp.dot(a, s2b_ref[...], preferred_element_type=jnp.float32)
        x3b = jnp.dot(b, s2b_ref[...], preferred_element_type=jnp.float32)
        x3_ref[0, pl.ds(0, _BH), :] = x3a
        x3_ref[0, pl.ds(_BH, _BH), :] = x3b
        la = jnp.dot(x3a.astype(jnp.bfloat16), wout,
                     preferred_element_type=jnp.float32)
        lb = jnp.dot(x3b.astype(jnp.bfloat16), wout,
                     preferred_element_type=jnp.float32)
        y_ref[0, pl.ds(0, _BH), :] = jax.nn.sigmoid(la)
        y_ref[0, pl.ds(_BH, _BH), :] = jax.nn.sigmoid(lb)


def kernel(x, adj, W1, W2, W_out):
    n, nfeat = x.shape
    nhid = W1.shape[1]
    nclass = W_out.shape[1]
    r = n // _BR

    x3, y = pl.pallas_call(
        _gcn_body,
        grid=(2, r),
        in_specs=[
            pl.BlockSpec((n, nfeat), lambda p, i: (0, 0)),
            pl.BlockSpec((_BH, n), lambda p, i: (2 * i, 0)),
            pl.BlockSpec((_BH, n), lambda p, i: (2 * i + 1, 0)),
            pl.BlockSpec((nfeat, nhid), lambda p, i: (0, 0)),
            pl.BlockSpec((nhid, nhid), lambda p, i: (0, 0)),
            pl.BlockSpec((nhid, nclass), lambda p, i: (0, 0)),
        ],
        out_specs=[
            pl.BlockSpec((1, _BR, nhid), lambda p, i: (p, i, 0)),
            pl.BlockSpec((1, _BR, nclass), lambda p, i: (p, i, 0)),
        ],
        out_shape=[
            jax.ShapeDtypeStruct((2, n, nhid), jnp.float32),
            jax.ShapeDtypeStruct((2, n, nclass), jnp.float32),
        ],
        scratch_shapes=[
            pltpu.VMEM((n, nhid), jnp.bfloat16),
            pltpu.VMEM((n, nhid), jnp.float32),
            pltpu.VMEM((n, nhid), jnp.bfloat16),
        ],
    )(x, adj, adj, W1, W2, W_out)

    return (y[1], x3[1])


# speedup vs baseline: 1.0459x; 1.0459x over previous
import jax
import jax.numpy as jnp
from jax import lax
from jax.experimental import pallas as pl
from jax.experimental.pallas import tpu as pltpu

_BR = 200
_NB = 3


def _gcn_body(x_ref, adj_ref, w1_ref, w2_ref, wout_ref,
              x3_ref, y_ref, abuf, s1_ref, s2_ref, s2b_ref, sem):
    n = x_ref.shape[0]
    r = n // _BR
    total = 2 * r

    s1 = jnp.dot(x_ref[...], w1_ref[...], preferred_element_type=jnp.float32)
    s1_ref[...] = s1.astype(jnp.bfloat16)
    w2 = w2_ref[...].astype(jnp.bfloat16)
    wout = wout_ref[...].astype(jnp.bfloat16)

    def block_copy(k, b):
        blk = lax.rem(k, r)
        return pltpu.make_async_copy(
            adj_ref.at[pl.ds(blk * _BR, _BR), :], abuf.at[b], sem.at[b])

    for b in range(_NB):
        block_copy(b, b).start()

    def super_step(s, carry):
        for b in range(_NB):
            k = s * _NB + b
            blk = lax.rem(k, r)

            @pl.when(k < total)
            def _():
                block_copy(k, b).wait()

            a = abuf[b].astype(jnp.bfloat16)

            @pl.when(k < r)
            def _():
                h = jnp.dot(a, s1_ref[...], preferred_element_type=jnp.float32)
                x1 = jnp.where(h >= 0, h, 0.01 * h)
                s2 = jnp.dot(x1.astype(jnp.bfloat16), w2,
                             preferred_element_type=jnp.float32)
                s2_ref[pl.ds(blk * _BR, _BR), :] = s2

            @pl.when(k == r)
            def _():
                s2b_ref[...] = s2_ref[...].astype(jnp.bfloat16)

            @pl.when(jnp.logical_and(k >= r, k < total))
            def _():
                x3 = jnp.dot(a, s2b_ref[...],
                             preferred_element_type=jnp.float32)
                x3_ref[pl.ds(blk * _BR, _BR), :] = x3
                logits = jnp.dot(x3.astype(jnp.bfloat16), wout,
                                 preferred_element_type=jnp.float32)
                y_ref[pl.ds(blk * _BR, _BR), :] = jax.nn.sigmoid(logits)

            @pl.when(k + _NB < total)
            def _():
                block_copy(k + _NB, b).start()
        return carry

    lax.fori_loop(0, pl.cdiv(total, _NB), super_step, 0)


def kernel(x, adj, W1, W2, W_out):
    n, nfeat = x.shape
    nhid = W1.shape[1]
    nclass = W_out.shape[1]

    x3, y = pl.pallas_call(
        _gcn_body,
        in_specs=[
            pl.BlockSpec(memory_space=pltpu.MemorySpace.VMEM),
            pl.BlockSpec(memory_space=pl.ANY),
            pl.BlockSpec(memory_space=pltpu.MemorySpace.VMEM),
            pl.BlockSpec(memory_space=pltpu.MemorySpace.VMEM),
            pl.BlockSpec(memory_space=pltpu.MemorySpace.VMEM),
        ],
        out_specs=[
            pl.BlockSpec(memory_space=pltpu.MemorySpace.VMEM),
            pl.BlockSpec(memory_space=pltpu.MemorySpace.VMEM),
        ],
        out_shape=[
            jax.ShapeDtypeStruct((n, nhid), jnp.float32),
            jax.ShapeDtypeStruct((n, nclass), jnp.float32),
        ],
        scratch_shapes=[
            pltpu.VMEM((_NB, _BR, n), jnp.float32),
            pltpu.VMEM((n, nhid), jnp.bfloat16),
            pltpu.VMEM((n, nhid), jnp.float32),
            pltpu.VMEM((n, nhid), jnp.bfloat16),
            pltpu.SemaphoreType.DMA((_NB,)),
        ],
    )(x, adj, W1, W2, W_out)

    return (y, x3)


# speedup vs baseline: 1.0495x; 1.0034x over previous
import jax
import jax.numpy as jnp
from jax import lax
from jax.experimental import pallas as pl
from jax.experimental.pallas import tpu as pltpu

_BR = 200
_NB = 4


def _gcn_body(x_ref, adj_ref, w1_ref, w2_ref, wout_ref,
              x3_ref, y_ref, abuf, s1_ref, s2_ref, sem):
    n = x_ref.shape[0]
    r = n // _BR
    total = 2 * r

    s1 = jnp.dot(x_ref[...], w1_ref[...], preferred_element_type=jnp.float32)
    s1_ref[...] = s1.astype(jnp.bfloat16)
    w2 = w2_ref[...].astype(jnp.bfloat16)
    wout = wout_ref[...].astype(jnp.bfloat16)

    def block_copy(k, b):
        blk = lax.rem(k, r)
        return pltpu.make_async_copy(
            adj_ref.at[pl.ds(blk * _BR, _BR), :], abuf.at[b], sem.at[b])

    for b in range(_NB):
        block_copy(b, b).start()

    def super_step(s, carry):
        for half in range(_NB // 2):
            b0 = 2 * half
            b1 = b0 + 1
            k0 = s * _NB + b0
            k1 = k0 + 1
            row = lax.rem(k0, r) * _BR

            block_copy(k0, b0).wait()
            block_copy(k1, b1).wait()

            @pl.when(k0 < r)
            def _():
                h0 = jnp.dot(abuf[b0].astype(jnp.bfloat16), s1_ref[...],
                             preferred_element_type=jnp.float32)
                h1 = jnp.dot(abuf[b1].astype(jnp.bfloat16), s1_ref[...],
                             preferred_element_type=jnp.float32)
                h = jnp.concatenate([h0, h1], axis=0)
                x1 = jnp.where(h >= 0, h, 0.01 * h)
                s2 = jnp.dot(x1.astype(jnp.bfloat16), w2,
                             preferred_element_type=jnp.float32)
                s2_ref[pl.ds(row, 2 * _BR), :] = s2.astype(jnp.bfloat16)

            @pl.when(k0 >= r)
            def _():
                x30 = jnp.dot(abuf[b0].astype(jnp.bfloat16), s2_ref[...],
                              preferred_element_type=jnp.float32)
                x31 = jnp.dot(abuf[b1].astype(jnp.bfloat16), s2_ref[...],
                              preferred_element_type=jnp.float32)
                x3 = jnp.concatenate([x30, x31], axis=0)
                x3_ref[pl.ds(row, 2 * _BR), :] = x3
                logits = jnp.dot(x3.astype(jnp.bfloat16), wout,
                                 preferred_element_type=jnp.float32)
                y_ref[pl.ds(row, 2 * _BR), :] = jax.nn.sigmoid(logits)

            @pl.when(k0 + _NB < total)
            def _():
                block_copy(k0 + _NB, b0).start()

            @pl.when(k1 + _NB < total)
            def _():
                block_copy(k1 + _NB, b1).start()
        return carry

    lax.fori_loop(0, total // _NB, super_step, 0)


def kernel(x, adj, W1, W2, W_out):
    n, nfeat = x.shape
    nhid = W1.shape[1]
    nclass = W_out.shape[1]

    x3, y = pl.pallas_call(
        _gcn_body,
        in_specs=[
            pl.BlockSpec(memory_space=pltpu.MemorySpace.VMEM),
            pl.BlockSpec(memory_space=pl.ANY),
            pl.BlockSpec(memory_space=pltpu.MemorySpace.VMEM),
            pl.BlockSpec(memory_space=pltpu.MemorySpace.VMEM),
            pl.BlockSpec(memory_space=pltpu.MemorySpace.VMEM),
        ],
        out_specs=[
            pl.BlockSpec(memory_space=pltpu.MemorySpace.VMEM),
            pl.BlockSpec(memory_space=pltpu.MemorySpace.VMEM),
        ],
        out_shape=[
            jax.ShapeDtypeStruct((n, nhid), jnp.float32),
            jax.ShapeDtypeStruct((n, nclass), jnp.float32),
        ],
        scratch_shapes=[
            pltpu.VMEM((_NB, _BR, n), jnp.float32),
            pltpu.VMEM((n, nhid), jnp.bfloat16),
            pltpu.VMEM((n, nhid), jnp.bfloat16),
            pltpu.SemaphoreType.DMA((_NB,)),
        ],
    )(x, adj, W1, W2, W_out)

    return (y, x3)
